# Initial kernel scaffold; baseline (speedup 1.0000x reference)
import jax
import jax.numpy as jnp
from jax.experimental import pallas as pl


def kernel(in_field, species_idx, shifts, scales):
    raise NotImplementedError("write your pallas kernel here")


# speedup vs baseline: 41.8970x; 41.8970x over previous
import dataclasses
import functools

import jax
import jax.numpy as jnp
from jax import lax
from jax.experimental import pallas as pl
from jax.experimental.pallas import tpu as pltpu
from jax.experimental.pallas import tpu_sc as plsc

_NC = 2
_NS = 16
_NW = _NC * _NS
_L = 16
_TPAD = 128


def _sc_affine_gather(x, idx, stab, ctab, *, chunk, tail):
    n = x.shape[0]
    mesh = plsc.VectorSubcoreMesh(core_axis_name="c", subcore_axis_name="s")
    cp = pltpu.CompilerParams()
    if "needs_layout_passes" in pltpu.CompilerParams.__dataclass_fields__:
        cp = dataclasses.replace(cp, needs_layout_passes=False)

    @functools.partial(
        pl.kernel,
        out_type=jax.ShapeDtypeStruct((n,), jnp.float32),
        mesh=mesh,
        compiler_params=cp,
        scratch_types=[
            pltpu.VMEM((chunk,), jnp.int32),
            pltpu.VMEM((chunk,), jnp.float32),
            pltpu.VMEM((chunk,), jnp.float32),
            pltpu.VMEM((_TPAD,), jnp.float32),
            pltpu.VMEM((_TPAD,), jnp.float32),
            pltpu.SemaphoreType.DMA,
            pltpu.SemaphoreType.DMA,
            pltpu.SemaphoreType.DMA,
            pltpu.SemaphoreType.DMA,
        ],
    )
    def body(x_hbm, idx_hbm, stab_hbm, ctab_hbm, out_hbm,
             idx_v, x_v, out_v, stab_v, ctab_v, sem0, sem1, sem2, sem3):
        wid = lax.axis_index("s") * _NC + lax.axis_index("c")
        base = wid * chunk

        def do_chunk(base, size):
            c0 = pltpu.async_copy(
                idx_hbm.at[pl.ds(base, size)], idx_v.at[pl.ds(0, size)], sem0)
            c1 = pltpu.async_copy(
                x_hbm.at[pl.ds(base, size)], x_v.at[pl.ds(0, size)], sem1)
            c2 = pltpu.async_copy(stab_hbm, stab_v, sem2)
            c3 = pltpu.async_copy(ctab_hbm, ctab_v, sem3)
            c0.wait()
            c1.wait()
            c2.wait()
            c3.wait()

            @pl.loop(0, size, step=_L)
            def _(c):
                sl = pl.ds(c, _L)
                iv = idx_v[sl]
                xv = x_v[sl]
                s = plsc.load_gather(stab_v, [iv])
                sc = plsc.load_gather(ctab_v, [iv])
                out_v[sl] = s + sc * xv

            pltpu.sync_copy(out_v.at[pl.ds(0, size)],
                            out_hbm.at[pl.ds(base, size)])

        @pl.when(wid < _NW - 1)
        def _():
            do_chunk(base, chunk)

        @pl.when(wid == _NW - 1)
        def _():
            do_chunk(base, tail)

    return body(x, idx, stab, ctab)


def kernel(in_field, species_idx, shifts, scales):
    n = in_field.shape[0]
    x = in_field.reshape(n).astype(jnp.float32)
    idx = species_idx.astype(jnp.int32)
    stab = jnp.zeros((_TPAD,), jnp.float32).at[: shifts.shape[0]].set(
        shifts.astype(jnp.float32))
    ctab = jnp.zeros((_TPAD,), jnp.float32).at[: scales.shape[0]].set(
        scales.astype(jnp.float32))

    chunk = ((n + _NW - 1) // _NW + _L - 1) // _L * _L
    tail = n - (_NW - 1) * chunk
    assert chunk % _L == 0 and tail % _L == 0 and tail > 0, (n, chunk, tail)

    out = _sc_affine_gather(x, idx, stab, ctab, chunk=chunk, tail=tail)
    return out.reshape(n, 1)


# speedup vs baseline: 44.7255x; 1.0675x over previous
import dataclasses
import functools

import jax
import jax.numpy as jnp
from jax import lax
from jax.experimental import pallas as pl
from jax.experimental.pallas import tpu as pltpu
from jax.experimental.pallas import tpu_sc as plsc

_NC = 2
_NS = 16
_NW = _NC * _NS
_L = 16
_UNROLL = 4


def _sc_affine_gather(x, idx, stab, ctab, *, chunk, tail):
    n = x.shape[0]
    mesh = plsc.VectorSubcoreMesh(core_axis_name="c", subcore_axis_name="s")
    cp = pltpu.CompilerParams()
    if "needs_layout_passes" in pltpu.CompilerParams.__dataclass_fields__:
        cp = dataclasses.replace(cp, needs_layout_passes=False)

    @functools.partial(
        pl.kernel,
        out_type=jax.ShapeDtypeStruct((n,), jnp.float32),
        mesh=mesh,
        compiler_params=cp,
        scratch_types=[
            pltpu.VMEM((chunk,), jnp.int32),
            pltpu.VMEM((chunk,), jnp.float32),
            pltpu.VMEM((chunk,), jnp.float32),
            pltpu.VMEM(stab.shape, jnp.float32),
            pltpu.VMEM(ctab.shape, jnp.float32),
            pltpu.SemaphoreType.DMA,
            pltpu.SemaphoreType.DMA,
            pltpu.SemaphoreType.DMA,
            pltpu.SemaphoreType.DMA,
        ],
    )
    def body(x_hbm, idx_hbm, stab_hbm, ctab_hbm, out_hbm,
             idx_v, x_v, out_v, stab_v, ctab_v, sem0, sem1, sem2, sem3):
        wid = lax.axis_index("s") * _NC + lax.axis_index("c")
        base = wid * chunk

        def do_chunk(base, size):
            c0 = pltpu.async_copy(
                idx_hbm.at[pl.ds(base, size)], idx_v.at[pl.ds(0, size)], sem0)
            c1 = pltpu.async_copy(
                x_hbm.at[pl.ds(base, size)], x_v.at[pl.ds(0, size)], sem1)
            c2 = pltpu.async_copy(stab_hbm, stab_v, sem2)
            c3 = pltpu.async_copy(ctab_hbm, ctab_v, sem3)
            c0.wait()
            c1.wait()
            c2.wait()
            c3.wait()

            def vec(c):
                sl = pl.ds(c, _L)
                iv = idx_v[sl]
                xv = x_v[sl]
                s = plsc.load_gather(stab_v, [iv])
                sc = plsc.load_gather(ctab_v, [iv])
                out_v[sl] = s + sc * xv

            main = size - size % (_L * _UNROLL)

            @pl.loop(0, main, step=_L * _UNROLL)
            def _(c):
                for u in range(_UNROLL):
                    vec(c + u * _L)

            @pl.loop(main, size, step=_L)
            def _(c):
                vec(c)

            pltpu.sync_copy(out_v.at[pl.ds(0, size)],
                            out_hbm.at[pl.ds(base, size)])

        @pl.when(wid < _NW - 1)
        def _():
            do_chunk(base, chunk)

        @pl.when(wid == _NW - 1)
        def _():
            do_chunk(base, tail)

    return body(x, idx, stab, ctab)


def kernel(in_field, species_idx, shifts, scales):
    n = in_field.shape[0]
    x = in_field.reshape(n).astype(jnp.float32)
    idx = species_idx.astype(jnp.int32)
    stab = shifts.astype(jnp.float32)
    ctab = scales.astype(jnp.float32)

    chunk = ((n + _NW - 1) // _NW + _L - 1) // _L * _L
    tail = n - (_NW - 1) * chunk
    assert chunk % _L == 0 and tail % _L == 0 and tail > 0, (n, chunk, tail)

    out = _sc_affine_gather(x, idx, stab, ctab, chunk=chunk, tail=tail)
    return out.reshape(n, 1)
